# baseline (speedup 1.0000x reference)
import jax
import jax.numpy as jnp
from jax.experimental import pallas as pl

_N = 1024
_C = 1000
_ROWS = 16384
_BR = 256


def _body(*refs):
    o_ref = refs[16]
    i = pl.program_id(0)
    lane = jax.lax.broadcasted_iota(jnp.int32, (_BR, 128), 1)

    pad7 = lane >= (_C - 7 * 128)
    K = [refs[g][...] for g in range(8)]
    K[7] = jnp.where(pad7, -jnp.inf, K[7])
    L = [refs[8 + g][...] for g in range(8)]
    L[7] = jnp.where(pad7, 0.0, L[7])
    V = [lane * 2 + (g * 256) + L[g].astype(jnp.int32)
         for g in range(8)]

    for ksz_log in range(1, 11):
        ksz = 1 << ksz_log
        for j_log in range(ksz_log - 1, -1, -1):
            j = 1 << j_log
            if j < 8:
                for g in range(8):
                    if g & j:
                        continue
                    h = g ^ j
                    a, b, va, vb = K[g], K[h], V[g], V[h]
                    after = a < b
                    if ksz < 8:
                        swap = (~after) if (g & ksz) else after
                    else:
                        asc = (lane & (ksz // 8)) == 0
                        swap = after == asc
                    K[g] = jnp.where(swap, b, a)
                    K[h] = jnp.where(swap, a, b)
                    V[g] = jnp.where(swap, vb, va)
                    V[h] = jnp.where(swap, va, vb)
            else:
                jl = j // 8
                asc = (lane & (ksz // 8)) == 0
                low = (lane & jl) == 0
                for g in range(8):
                    k, v = K[g], V[g]
                    pk = jnp.where(low, jnp.roll(k, -jl, axis=1),
                                   jnp.roll(k, jl, axis=1))
                    pv = jnp.where(low, jnp.roll(v, -jl, axis=1),
                                   jnp.roll(v, jl, axis=1))
                    after = k < pk
                    take = after == (asc == low)
                    K[g] = jnp.where(take, pk, k)
                    V[g] = jnp.where(take, pv, v)

    ls = [(V[g] & 1).astype(jnp.float32) for g in range(8)]
    run = ls[0]
    cums = [run]
    for g in range(1, 8):
        run = run + ls[g]
        cums.append(run)
    inc = run
    for d in (1, 2, 4, 8, 16, 32, 64):
        inc = inc + jnp.where(lane >= d, jnp.roll(inc, d, axis=1), 0.0)
    excl = inc - run

    term_sum = jnp.zeros_like(run)
    npos = run * 0.0
    for g in range(8):
        pos = (lane * 8 + g + 1).astype(jnp.float32)
        cum = cums[g] + excl
        term_sum = term_sum + ls[g] * (cum / pos)
        npos = npos + ls[g]
    row_sum = jnp.sum(term_sum, axis=1)
    npos_row = jnp.sum(npos, axis=1)
    acc = jnp.sum(row_sum / npos_row)

    contrib = jnp.reshape(acc * (1.0 / _ROWS), (1, 1))
    @pl.when(i == 0)
    def _():
        o_ref[...] = contrib
    @pl.when(i > 0)
    def _():
        o_ref[...] = o_ref[...] + contrib


def kernel(preds, labels):
    nb = _ROWS // _BR
    tile_specs = [
        pl.BlockSpec((_BR, 128), (lambda i, g=g: (i, g))) for g in range(8)
    ]
    out = pl.pallas_call(
        _body,
        grid=(nb,),
        in_specs=tile_specs + tile_specs,
        out_specs=pl.BlockSpec((1, 1), lambda i: (0, 0)),
        out_shape=jax.ShapeDtypeStruct((1, 1), jnp.float32),
    )(*([preds] * 8 + [labels] * 8))
    return out[0, 0]
